# baseline (speedup 1.0000x reference)
import functools

import jax
import jax.numpy as jnp
from jax import lax
from jax.experimental import pallas as pl
from jax.experimental.pallas import tpu as pltpu
from jax.experimental.pallas import tpu_sc as plsc

_P = 8
_NBUF = 3


def _sc_add(input_hbm, pe_hbm, out_hbm,
            in0, in1, in2, pe0, pe1, pe2,
            si0, si1, si2, sp0, sp1, sp2, so0, so1, so2):
    S, B, D = input_hbm.shape
    info = plsc.get_sparse_core_info()
    nw = info.num_cores * info.num_subcores
    wid = lax.axis_index("s") * info.num_cores + lax.axis_index("c")
    pos_per_w = S // nw
    n_chunks = pos_per_w // _P
    pos0 = wid * pos_per_w
    lanes = info.num_lanes

    in_bufs = (in0, in1, in2)
    pe_bufs = (pe0, pe1, pe2)
    in_sems = (si0, si1, si2)
    pe_sems = (sp0, sp1, sp2)
    out_sems = (so0, so1, so2)

    def in_copies(ci):
        k = ci % _NBUF
        base = pos0 + ci * _P
        return (
            pltpu.make_async_copy(input_hbm.at[pl.ds(base, _P)], in_bufs[k], in_sems[k]),
            pltpu.make_async_copy(pe_hbm.at[pl.ds(base, _P)], pe_bufs[k], pe_sems[k]),
        )

    def out_copy(ci):
        k = ci % _NBUF
        base = pos0 + ci * _P
        return pltpu.make_async_copy(in_bufs[k], out_hbm.at[pl.ds(base, _P)], out_sems[k])

    def compute(k):
        in_buf, pe_buf = in_bufs[k], pe_bufs[k]

        def jloop(j, carry):
            for p in range(_P):
                pe_v = pe_buf[p, pl.ds(j * lanes, lanes)]
                for b in range(B):
                    plsc.addupdate(in_buf.at[p, b, pl.ds(j * lanes, lanes)], pe_v)
            return carry

        lax.fori_loop(0, D // lanes, jloop, 0)

    for c in in_copies(0):
        c.start()

    for ci in range(n_chunks):
        if ci >= 2:
            out_copy(ci - 2).wait()
        if ci + 1 < n_chunks:
            for c in in_copies(ci + 1):
                c.start()
        for c in in_copies(ci):
            c.wait()
        compute(ci % _NBUF)
        out_copy(ci).start()

    out_copy(n_chunks - 2).wait()
    out_copy(n_chunks - 1).wait()


def kernel(input, pe_table):
    S, B, D = input.shape
    mesh = plsc.VectorSubcoreMesh(core_axis_name="c", subcore_axis_name="s")
    f = functools.partial(
        pl.kernel,
        mesh=mesh,
        out_type=jax.ShapeDtypeStruct((S, B, D), input.dtype),
        scratch_types=[
            pltpu.VMEM((_P, B, D), jnp.float32),
            pltpu.VMEM((_P, B, D), jnp.float32),
            pltpu.VMEM((_P, B, D), jnp.float32),
            pltpu.VMEM((_P, D), jnp.float32),
            pltpu.VMEM((_P, D), jnp.float32),
            pltpu.VMEM((_P, D), jnp.float32),
            pltpu.SemaphoreType.DMA,
            pltpu.SemaphoreType.DMA,
            pltpu.SemaphoreType.DMA,
            pltpu.SemaphoreType.DMA,
            pltpu.SemaphoreType.DMA,
            pltpu.SemaphoreType.DMA,
            pltpu.SemaphoreType.DMA,
            pltpu.SemaphoreType.DMA,
            pltpu.SemaphoreType.DMA,
        ],
    )(_sc_add)
    return f(input, pe_table)


# speedup vs baseline: 1.0822x; 1.0822x over previous
import functools

import jax
import jax.numpy as jnp
from jax import lax
from jax.experimental import pallas as pl
from jax.experimental.pallas import tpu as pltpu
from jax.experimental.pallas import tpu_sc as plsc

_P = 8
_NBUF = 3
_S_SC = 512
_BS = 512


def _sc_add(input_hbm, pe_hbm, out_hbm,
            in0, in1, in2, pe0, pe1, pe2,
            si0, si1, si2, sp0, sp1, sp2, so0, so1, so2):
    S, B, D = input_hbm.shape
    info = plsc.get_sparse_core_info()
    nw = info.num_cores * info.num_subcores
    wid = lax.axis_index("s") * info.num_cores + lax.axis_index("c")
    pos_per_w = _S_SC // nw
    n_chunks = pos_per_w // _P
    pos0 = (S - _S_SC) + wid * pos_per_w
    lanes = info.num_lanes

    in_bufs = (in0, in1, in2)
    pe_bufs = (pe0, pe1, pe2)
    in_sems = (si0, si1, si2)
    pe_sems = (sp0, sp1, sp2)
    out_sems = (so0, so1, so2)

    def in_copies(ci):
        k = ci % _NBUF
        base = pos0 + ci * _P
        return (
            pltpu.make_async_copy(input_hbm.at[pl.ds(base, _P)], in_bufs[k], in_sems[k]),
            pltpu.make_async_copy(pe_hbm.at[pl.ds(base, _P)], pe_bufs[k], pe_sems[k]),
        )

    def out_copy(ci):
        k = ci % _NBUF
        base = pos0 + ci * _P
        return pltpu.make_async_copy(in_bufs[k], out_hbm.at[pl.ds(base, _P)], out_sems[k])

    def compute(k):
        in_buf, pe_buf = in_bufs[k], pe_bufs[k]

        def jloop(j, carry):
            for p in range(_P):
                pe_v = pe_buf[p, pl.ds(j * lanes, lanes)]
                for b in range(B):
                    plsc.addupdate(in_buf.at[p, b, pl.ds(j * lanes, lanes)], pe_v)
            return carry

        lax.fori_loop(0, D // lanes, jloop, 0)

    for c in in_copies(0):
        c.start()

    for ci in range(n_chunks):
        if ci >= 2:
            out_copy(ci - 2).wait()
        if ci + 1 < n_chunks:
            for c in in_copies(ci + 1):
                c.start()
        for c in in_copies(ci):
            c.wait()
        compute(ci % _NBUF)
        out_copy(ci).start()

    if n_chunks >= 2:
        out_copy(n_chunks - 2).wait()
    out_copy(n_chunks - 1).wait()


def _sc_call(input, pe_table):
    S, B, D = input.shape
    mesh = plsc.VectorSubcoreMesh(core_axis_name="c", subcore_axis_name="s")
    f = functools.partial(
        pl.kernel,
        mesh=mesh,
        out_type=jax.ShapeDtypeStruct((S, B, D), input.dtype),
        scratch_types=[
            pltpu.VMEM((_P, B, D), jnp.float32),
            pltpu.VMEM((_P, B, D), jnp.float32),
            pltpu.VMEM((_P, B, D), jnp.float32),
            pltpu.VMEM((_P, D), jnp.float32),
            pltpu.VMEM((_P, D), jnp.float32),
            pltpu.VMEM((_P, D), jnp.float32),
            pltpu.SemaphoreType.DMA,
            pltpu.SemaphoreType.DMA,
            pltpu.SemaphoreType.DMA,
            pltpu.SemaphoreType.DMA,
            pltpu.SemaphoreType.DMA,
            pltpu.SemaphoreType.DMA,
            pltpu.SemaphoreType.DMA,
            pltpu.SemaphoreType.DMA,
            pltpu.SemaphoreType.DMA,
        ],
    )(_sc_add)
    return f(input, pe_table)


def _tc_body(in_ref, pe_ref, _alias_ref, out_ref):
    out_ref[...] = in_ref[...] + pe_ref[...][:, None, :]


def kernel(input, pe_table):
    S, B, D = input.shape
    partial = _sc_call(input, pe_table)
    grid = ((S - _S_SC) // _BS,)
    return pl.pallas_call(
        _tc_body,
        grid=grid,
        in_specs=[
            pl.BlockSpec((_BS, B, D), lambda i: (i, 0, 0)),
            pl.BlockSpec((_BS, D), lambda i: (i, 0)),
            pl.BlockSpec(memory_space=pl.ANY),
        ],
        out_specs=pl.BlockSpec((_BS, B, D), lambda i: (i, 0, 0)),
        out_shape=jax.ShapeDtypeStruct((S, B, D), input.dtype),
        input_output_aliases={2: 0},
        compiler_params=pltpu.CompilerParams(
            dimension_semantics=("arbitrary",),
        ),
    )(input, pe_table, partial)
